# Initial kernel scaffold; baseline (speedup 1.0000x reference)
import jax
import jax.numpy as jnp
from jax.experimental import pallas as pl


def kernel(x_0, x_1, pos, cell_ind_0, cell_ind_1, adj_0_0, adj_0_1_send, adj_0_1_recv, adj_1_1, params):
    raise NotImplementedError("write your pallas kernel here")


# speedup vs baseline: 3.3432x; 3.3432x over previous
import functools

import jax
import jax.numpy as jnp
from jax import lax
from jax.experimental import pallas as pl
from jax.experimental.pallas import tpu as pltpu
from jax.experimental.pallas import tpu_sc as plsc

f32 = jnp.float32
i32 = jnp.int32

N0 = 10000
N0P = 10240
E = 160000
EP = 163840
H = 64
NC, NS = 2, 16
NW = NC * NS
EPW = EP // NW
EHALF = EP // NC
GC = 1024
RPT = N0P // NS
BLKE = 4096

_mesh = plsc.VectorSubcoreMesh(core_axis_name="c", subcore_axis_name="s")


def _silu(x):
    return x * jax.nn.sigmoid(x)


def _make_gather2(width):
    @functools.partial(
        pl.kernel,
        out_type=(jax.ShapeDtypeStruct((EP, width), f32),
                  jax.ShapeDtypeStruct((EP, width), f32)),
        mesh=_mesh,
        scratch_types=[
            pltpu.VMEM((GC,), i32),
            pltpu.VMEM((GC, width), f32),
            pltpu.SemaphoreType.DMA,
        ],
        compiler_params=pltpu.CompilerParams(use_tc_tiling_on_sc=False),
    )
    def _g2(h_h, s_h, r_h, xs_h, xr_h, iv, rows, sem):
        wid = lax.axis_index("s") * NC + lax.axis_index("c")
        base = wid * EPW

        def chunk_s(j, carry):
            off = base + j * GC
            pltpu.sync_copy(s_h.at[pl.ds(off, GC)], iv)
            pltpu.async_copy(h_h.at[iv], rows, sem).wait()
            pltpu.sync_copy(rows, xs_h.at[pl.ds(off, GC)])
            return carry

        def chunk_r(j, carry):
            off = base + j * GC
            pltpu.sync_copy(r_h.at[pl.ds(off, GC)], iv)
            pltpu.async_copy(h_h.at[iv], rows, sem).wait()
            pltpu.sync_copy(rows, xr_h.at[pl.ds(off, GC)])
            return carry

        lax.fori_loop(0, EPW // GC, chunk_s, 0)
        lax.fori_loop(0, EPW // GC, chunk_r, 0)

    return _g2


_sc_gather2 = _make_gather2(H)
_sc_gather2_pos = _make_gather2(16)


@functools.partial(
    pl.kernel,
    out_type=jax.ShapeDtypeStruct((NC, N0P, H), f32),
    mesh=_mesh,
    scratch_types=[
        pltpu.VMEM_SHARED((N0P, H), f32),
        pltpu.VMEM((GC,), i32),
        pltpu.VMEM((GC, H), f32),
    ],
    compiler_params=pltpu.CompilerParams(use_tc_tiling_on_sc=False),
)
def _sc_scatter(m_h, r_h, z_h, o_h, shared, iv, mv):
    c = lax.axis_index("c")
    s = lax.axis_index("s")
    pltpu.sync_copy(z_h.at[pl.ds(s * RPT, RPT)], shared.at[pl.ds(s * RPT, RPT)])
    plsc.subcore_barrier()
    ebase = c * EHALF + s * EPW

    def chunk(j, carry):
        off = ebase + j * GC
        pltpu.sync_copy(r_h.at[pl.ds(off, GC)], iv)
        pltpu.sync_copy(m_h.at[pl.ds(off, GC)], mv)
        pltpu.sync_copy(mv, shared.at[iv], add=True)
        return carry

    lax.fori_loop(0, EPW // GC, chunk, 0)
    plsc.subcore_barrier()
    pltpu.sync_copy(shared.at[pl.ds(s * RPT, RPT)],
                    o_h.at[c, pl.ds(s * RPT, RPT)])


def _emb_body(x_ref, w_ref, b_ref, o_ref):
    o_ref[...] = jnp.dot(x_ref[...], w_ref[...],
                         preferred_element_type=f32) + b_ref[...]


def _tc_emb(x, w, b):
    return pl.pallas_call(
        _emb_body,
        out_shape=jax.ShapeDtypeStruct((N0P, H), f32),
    )(x, w, b[None, :])


def _d2_body(ps_ref, pr_ref, o_ref):
    diff = ps_ref[...] - pr_ref[...]
    sq = diff * diff
    o_ref[...] = jnp.sqrt(
        jnp.sum(sq[:, 0:3], axis=1, keepdims=True))


def _tc_d2(ps, pr):
    n = EP // BLKE
    pblk = lambda: pl.BlockSpec((BLKE, 16), lambda i: (i, 0))
    return pl.pallas_call(
        _d2_body,
        grid=(n,),
        in_specs=[pblk(), pblk()],
        out_specs=pl.BlockSpec((BLKE, 1), lambda i: (i, 0)),
        out_shape=jax.ShapeDtypeStruct((EP, 1), f32),
    )(ps, pr)


def _bn_body(d_ref, s1_ref, s2_ref):
    dd = d_ref[...]
    s1_ref[...] = jnp.sum(dd, axis=0, keepdims=True)
    s2_ref[...] = jnp.sum(dd * dd, axis=0, keepdims=True)


def _tc_bn(d):
    return pl.pallas_call(
        _bn_body,
        out_shape=(jax.ShapeDtypeStruct((1, 128), f32),
                   jax.ShapeDtypeStruct((1, 128), f32)),
    )(d)


def _msg_body(xs_ref, xr_ref, d_ref, a_ref, b_ref, cp_ref, b1_ref,
              w2_ref, b2_ref, o_ref):
    pre = (jnp.dot(xs_ref[...], a_ref[...], preferred_element_type=f32)
           + jnp.dot(xr_ref[...], b_ref[...], preferred_element_type=f32)
           + d_ref[...] * cp_ref[...] + b1_ref[...])
    h1 = _silu(pre)
    o_ref[...] = _silu(jnp.dot(h1, w2_ref[...],
                               preferred_element_type=f32) + b2_ref[...])


def _tc_msg(xs, xr, dcol, a, b, cp, b1, w2, b2):
    n = EP // BLKE
    eblk = lambda: pl.BlockSpec((BLKE, H), lambda i: (i, 0))
    wblk = lambda r: pl.BlockSpec((r, H), lambda i: (0, 0))
    return pl.pallas_call(
        _msg_body,
        grid=(n,),
        in_specs=[eblk(), eblk(), pl.BlockSpec((BLKE, 1), lambda i: (i, 0)),
                  wblk(H), wblk(H), wblk(1), wblk(1), wblk(H), wblk(1)],
        out_specs=eblk(),
        out_shape=jax.ShapeDtypeStruct((EP, H), f32),
    )(xs, xr, dcol, a, b, cp[None, :], b1[None, :], w2, b2[None, :])


def _upd_body(h_ref, a0_ref, a1_ref, u1x_ref, u1a_ref, bu1_ref,
              u2_ref, bu2_ref, o_ref):
    hh = h_ref[...]
    agg = a0_ref[...] + a1_ref[...]
    u = _silu(jnp.dot(hh, u1x_ref[...], preferred_element_type=f32)
              + jnp.dot(agg, u1a_ref[...], preferred_element_type=f32)
              + bu1_ref[...])
    o_ref[...] = hh + _silu(jnp.dot(u, u2_ref[...],
                                    preferred_element_type=f32) + bu2_ref[...])


def _tc_upd(h, a0, a1, u1x, u1a, bu1, u2, bu2):
    return pl.pallas_call(
        _upd_body,
        out_shape=jax.ShapeDtypeStruct((N0P, H), f32),
    )(h, a0, a1, u1x, u1a, bu1[None, :], u2, bu2[None, :])


def _upd_final_body(h_ref, a0_ref, a1_ref, u1x_ref, u1a_ref, bu1_ref,
                    u2_ref, bu2_ref, p_ref, bp_ref, o_ref):
    hh = h_ref[...]
    agg = a0_ref[...] + a1_ref[...]
    u = _silu(jnp.dot(hh, u1x_ref[...], preferred_element_type=f32)
              + jnp.dot(agg, u1a_ref[...], preferred_element_type=f32)
              + bu1_ref[...])
    h2 = hh + _silu(jnp.dot(u, u2_ref[...],
                            preferred_element_type=f32) + bu2_ref[...])
    o_ref[...] = jnp.dot(h2, p_ref[...],
                         preferred_element_type=f32) + bp_ref[...]


def _tc_upd_final(h, a0, a1, u1x, u1a, bu1, u2, bu2, p, bp):
    return pl.pallas_call(
        _upd_final_body,
        out_shape=jax.ShapeDtypeStruct((N0P, H), f32),
    )(h, a0, a1, u1x, u1a, bu1[None, :], u2, bu2[None, :], p, bp[None, :])


def kernel(x_0, x_1, pos, cell_ind_0, cell_ind_1, adj_0_0, adj_0_1_send,
           adj_0_1_recv, adj_1_1, params):
    send = adj_0_0[0].astype(i32)
    recv = adj_0_0[1].astype(i32)
    pad = EP - E
    zi = jnp.zeros((pad,), i32)
    send_p = jnp.concatenate([send, zi])
    recv_p = jnp.concatenate([recv, zi])
    recv_sc = jnp.concatenate([recv, N0 + (jnp.arange(pad, dtype=i32) % 8)])

    posrow = jnp.pad(pos, ((0, N0P - N0), (0, 13)))
    x0p = jnp.pad(x_0, ((0, N0P - N0), (0, 0)))

    h = _tc_emb(x0p, params["emb"]["0"]["W"], params["emb"]["0"]["b"])

    ps, pr = _sc_gather2_pos(posrow, send_p, recv_p)
    dcol = _tc_d2(ps, pr)
    s1, s2 = _tc_bn(dcol.reshape(EP // 128, 128))
    mu = jnp.sum(s1) / E
    var = jnp.sum(s2) / E - mu * mu
    sinv = lax.rsqrt(var + 1e-5)

    zeros_n = jnp.zeros((N0P, H), f32)

    nlayers = len(params["layers"])
    for li, layer in enumerate(params["layers"]):
        w1 = layer["msg"]["0_0"]["l1"]["W"]
        b1 = layer["msg"]["0_0"]["l1"]["b"]
        a_w, b_w, c_w = w1[:H], w1[H:2 * H], w1[2 * H:]
        ctil = jnp.sum(c_w, axis=0)
        cp = ctil * sinv
        b1p = b1 - mu * sinv * ctil
        w2 = layer["msg"]["0_0"]["l2"]["W"]
        b2 = layer["msg"]["0_0"]["l2"]["b"]

        xs, xr = _sc_gather2(h, send_p, recv_p)
        m = _tc_msg(xs, xr, dcol, a_w, b_w, cp, b1p, w2, b2)
        aggp = _sc_scatter(m, recv_sc, zeros_n)

        u1 = layer["upd"]["0"]["l1"]["W"]
        bu1 = layer["upd"]["0"]["l1"]["b"]
        u2 = layer["upd"]["0"]["l2"]["W"]
        bu2 = layer["upd"]["0"]["l2"]["b"]
        if li == nlayers - 1:
            h = _tc_upd_final(h, aggp[0], aggp[1], u1[:H], u1[H:], bu1,
                              u2, bu2, params["pre_pool"]["0"]["W"],
                              params["pre_pool"]["0"]["b"])
        else:
            h = _tc_upd(h, aggp[0], aggp[1], u1[:H], u1[H:], bu1, u2, bu2)

    out = h[:N0]
    return (out, out, jnp.zeros((1, H), f32))


# speedup vs baseline: 4.8685x; 1.4562x over previous
import functools

import jax
import jax.numpy as jnp
from jax import lax
from jax.experimental import pallas as pl
from jax.experimental.pallas import tpu as pltpu
from jax.experimental.pallas import tpu_sc as plsc

f32 = jnp.float32
i32 = jnp.int32

N0 = 10000
N0P = 10240
E = 160000
EP = 163840
H = 64
NC, NS = 2, 16
NW = NC * NS
EPW = EP // NW
EHALF = EP // NC
GC = 1024
RPT = N0P // NS
BLKE = 4096

_mesh = plsc.VectorSubcoreMesh(core_axis_name="c", subcore_axis_name="s")


def _silu(x):
    return x * jax.nn.sigmoid(x)


GC2 = 512
NCH = EPW // GC2


def _make_gather2(width):
    @functools.partial(
        pl.kernel,
        out_type=(jax.ShapeDtypeStruct((EP, width), f32),
                  jax.ShapeDtypeStruct((EP, width), f32)),
        mesh=_mesh,
        scratch_types=[
            pltpu.VMEM_SHARED((N0P, width), f32),
            pltpu.VMEM((EPW,), i32),
            pltpu.VMEM((EPW,), i32),
            pltpu.VMEM((GC2, width), f32),
            pltpu.VMEM((GC2, width), f32),
            pltpu.SemaphoreType.DMA,
            pltpu.SemaphoreType.DMA,
            pltpu.SemaphoreType.DMA,
            pltpu.SemaphoreType.DMA,
        ],
        compiler_params=pltpu.CompilerParams(use_tc_tiling_on_sc=False),
    )
    def _g2(h_h, s_h, r_h, xs_h, xr_h, shared, sv, rv, row0, row1,
            g0, g1, w0, w1):
        s = lax.axis_index("s")
        wid = s * NC + lax.axis_index("c")
        base = wid * EPW
        pltpu.sync_copy(h_h.at[pl.ds(s * RPT, RPT)],
                        shared.at[pl.ds(s * RPT, RPT)])
        pltpu.sync_copy(s_h.at[pl.ds(base, EPW)], sv)
        pltpu.sync_copy(r_h.at[pl.ds(base, EPW)], rv)
        plsc.subcore_barrier()

        rows = (row0, row1)
        gsems = (g0, g1)
        wsems = (w0, w1)
        nsteps = 2 * NCH
        wdesc = [None] * nsteps
        for k in range(nsteps):
            b = k % 2
            if k >= 2:
                wdesc[k - 2].wait()
            iv = sv if k < NCH else rv
            oh = xs_h if k < NCH else xr_h
            j = k % NCH
            pltpu.async_copy(shared.at[iv.at[pl.ds(j * GC2, GC2)]],
                             rows[b], gsems[b]).wait()
            wdesc[k] = pltpu.async_copy(
                rows[b], oh.at[pl.ds(base + j * GC2, GC2)], wsems[b])
        wdesc[nsteps - 2].wait()
        wdesc[nsteps - 1].wait()

    return _g2


_sc_gather2 = _make_gather2(H)
_sc_gather2_pos = _make_gather2(16)


@functools.partial(
    pl.kernel,
    out_type=jax.ShapeDtypeStruct((NC, N0P, H), f32),
    mesh=_mesh,
    scratch_types=[
        pltpu.VMEM_SHARED((N0P, H), f32),
        pltpu.VMEM((GC,), i32),
        pltpu.VMEM((GC, H), f32),
    ],
    compiler_params=pltpu.CompilerParams(use_tc_tiling_on_sc=False),
)
def _sc_scatter(m_h, r_h, z_h, o_h, shared, iv, mv):
    c = lax.axis_index("c")
    s = lax.axis_index("s")
    pltpu.sync_copy(z_h.at[pl.ds(s * RPT, RPT)], shared.at[pl.ds(s * RPT, RPT)])
    plsc.subcore_barrier()
    ebase = c * EHALF + s * EPW

    def chunk(j, carry):
        off = ebase + j * GC
        pltpu.sync_copy(r_h.at[pl.ds(off, GC)], iv)
        pltpu.sync_copy(m_h.at[pl.ds(off, GC)], mv)
        pltpu.sync_copy(mv, shared.at[iv], add=True)
        return carry

    lax.fori_loop(0, EPW // GC, chunk, 0)
    plsc.subcore_barrier()
    pltpu.sync_copy(shared.at[pl.ds(s * RPT, RPT)],
                    o_h.at[c, pl.ds(s * RPT, RPT)])


def _emb_body(x_ref, w_ref, b_ref, o_ref):
    o_ref[...] = jnp.dot(x_ref[...], w_ref[...],
                         preferred_element_type=f32) + b_ref[...]


def _tc_emb(x, w, b):
    return pl.pallas_call(
        _emb_body,
        out_shape=jax.ShapeDtypeStruct((N0P, H), f32),
    )(x, w, b[None, :])


def _d2_body(ps_ref, pr_ref, o_ref):
    diff = ps_ref[...] - pr_ref[...]
    sq = diff * diff
    o_ref[...] = jnp.sqrt(
        jnp.sum(sq[:, 0:3], axis=1, keepdims=True))


def _tc_d2(ps, pr):
    n = EP // BLKE
    pblk = lambda: pl.BlockSpec((BLKE, 16), lambda i: (i, 0))
    return pl.pallas_call(
        _d2_body,
        grid=(n,),
        in_specs=[pblk(), pblk()],
        out_specs=pl.BlockSpec((BLKE, 1), lambda i: (i, 0)),
        out_shape=jax.ShapeDtypeStruct((EP, 1), f32),
    )(ps, pr)


def _bn_body(d_ref, s1_ref, s2_ref):
    dd = d_ref[...]
    s1_ref[...] = jnp.sum(dd, axis=0, keepdims=True)
    s2_ref[...] = jnp.sum(dd * dd, axis=0, keepdims=True)


def _tc_bn(d):
    return pl.pallas_call(
        _bn_body,
        out_shape=(jax.ShapeDtypeStruct((1, 128), f32),
                   jax.ShapeDtypeStruct((1, 128), f32)),
    )(d)


def _msg_body(xs_ref, xr_ref, d_ref, a_ref, b_ref, cp_ref, b1_ref,
              w2_ref, b2_ref, o_ref):
    pre = (jnp.dot(xs_ref[...], a_ref[...], preferred_element_type=f32)
           + jnp.dot(xr_ref[...], b_ref[...], preferred_element_type=f32)
           + d_ref[...] * cp_ref[...] + b1_ref[...])
    h1 = _silu(pre)
    o_ref[...] = _silu(jnp.dot(h1, w2_ref[...],
                               preferred_element_type=f32) + b2_ref[...])


def _tc_msg(xs, xr, dcol, a, b, cp, b1, w2, b2):
    n = EP // BLKE
    eblk = lambda: pl.BlockSpec((BLKE, H), lambda i: (i, 0))
    wblk = lambda r: pl.BlockSpec((r, H), lambda i: (0, 0))
    return pl.pallas_call(
        _msg_body,
        grid=(n,),
        in_specs=[eblk(), eblk(), pl.BlockSpec((BLKE, 1), lambda i: (i, 0)),
                  wblk(H), wblk(H), wblk(1), wblk(1), wblk(H), wblk(1)],
        out_specs=eblk(),
        out_shape=jax.ShapeDtypeStruct((EP, H), f32),
    )(xs, xr, dcol, a, b, cp[None, :], b1[None, :], w2, b2[None, :])


def _upd_body(h_ref, a0_ref, a1_ref, u1x_ref, u1a_ref, bu1_ref,
              u2_ref, bu2_ref, o_ref):
    hh = h_ref[...]
    agg = a0_ref[...] + a1_ref[...]
    u = _silu(jnp.dot(hh, u1x_ref[...], preferred_element_type=f32)
              + jnp.dot(agg, u1a_ref[...], preferred_element_type=f32)
              + bu1_ref[...])
    o_ref[...] = hh + _silu(jnp.dot(u, u2_ref[...],
                                    preferred_element_type=f32) + bu2_ref[...])


def _tc_upd(h, a0, a1, u1x, u1a, bu1, u2, bu2):
    return pl.pallas_call(
        _upd_body,
        out_shape=jax.ShapeDtypeStruct((N0P, H), f32),
    )(h, a0, a1, u1x, u1a, bu1[None, :], u2, bu2[None, :])


def _upd_final_body(h_ref, a0_ref, a1_ref, u1x_ref, u1a_ref, bu1_ref,
                    u2_ref, bu2_ref, p_ref, bp_ref, o_ref):
    hh = h_ref[...]
    agg = a0_ref[...] + a1_ref[...]
    u = _silu(jnp.dot(hh, u1x_ref[...], preferred_element_type=f32)
              + jnp.dot(agg, u1a_ref[...], preferred_element_type=f32)
              + bu1_ref[...])
    h2 = hh + _silu(jnp.dot(u, u2_ref[...],
                            preferred_element_type=f32) + bu2_ref[...])
    o_ref[...] = jnp.dot(h2, p_ref[...],
                         preferred_element_type=f32) + bp_ref[...]


def _tc_upd_final(h, a0, a1, u1x, u1a, bu1, u2, bu2, p, bp):
    return pl.pallas_call(
        _upd_final_body,
        out_shape=jax.ShapeDtypeStruct((N0P, H), f32),
    )(h, a0, a1, u1x, u1a, bu1[None, :], u2, bu2[None, :], p, bp[None, :])


def kernel(x_0, x_1, pos, cell_ind_0, cell_ind_1, adj_0_0, adj_0_1_send,
           adj_0_1_recv, adj_1_1, params):
    send = adj_0_0[0].astype(i32)
    recv = adj_0_0[1].astype(i32)
    pad = EP - E
    zi = jnp.zeros((pad,), i32)
    send_p = jnp.concatenate([send, zi])
    recv_p = jnp.concatenate([recv, zi])
    recv_sc = jnp.concatenate([recv, N0 + (jnp.arange(pad, dtype=i32) % 8)])

    posrow = jnp.pad(pos, ((0, N0P - N0), (0, 13)))
    x0p = jnp.pad(x_0, ((0, N0P - N0), (0, 0)))

    h = _tc_emb(x0p, params["emb"]["0"]["W"], params["emb"]["0"]["b"])

    ps, pr = _sc_gather2_pos(posrow, send_p, recv_p)
    dcol = _tc_d2(ps, pr)
    s1, s2 = _tc_bn(dcol.reshape(EP // 128, 128))
    mu = jnp.sum(s1) / E
    var = jnp.sum(s2) / E - mu * mu
    sinv = lax.rsqrt(var + 1e-5)

    zeros_n = jnp.zeros((N0P, H), f32)

    nlayers = len(params["layers"])
    for li, layer in enumerate(params["layers"]):
        w1 = layer["msg"]["0_0"]["l1"]["W"]
        b1 = layer["msg"]["0_0"]["l1"]["b"]
        a_w, b_w, c_w = w1[:H], w1[H:2 * H], w1[2 * H:]
        ctil = jnp.sum(c_w, axis=0)
        cp = ctil * sinv
        b1p = b1 - mu * sinv * ctil
        w2 = layer["msg"]["0_0"]["l2"]["W"]
        b2 = layer["msg"]["0_0"]["l2"]["b"]

        xs, xr = _sc_gather2(h, send_p, recv_p)
        m = _tc_msg(xs, xr, dcol, a_w, b_w, cp, b1p, w2, b2)
        aggp = _sc_scatter(m, recv_sc, zeros_n)

        u1 = layer["upd"]["0"]["l1"]["W"]
        bu1 = layer["upd"]["0"]["l1"]["b"]
        u2 = layer["upd"]["0"]["l2"]["W"]
        bu2 = layer["upd"]["0"]["l2"]["b"]
        if li == nlayers - 1:
            h = _tc_upd_final(h, aggp[0], aggp[1], u1[:H], u1[H:], bu1,
                              u2, bu2, params["pre_pool"]["0"]["W"],
                              params["pre_pool"]["0"]["b"])
        else:
            h = _tc_upd(h, aggp[0], aggp[1], u1[:H], u1[H:], bu1, u2, bu2)

    out = h[:N0]
    return (out, out, jnp.zeros((1, H), f32))


# speedup vs baseline: 9.8191x; 2.0169x over previous
import functools

import numpy as np

import jax
import jax.numpy as jnp
from jax import lax
from jax.experimental import pallas as pl
from jax.experimental.pallas import tpu as pltpu
from jax.experimental.pallas import tpu_sc as plsc

f32 = jnp.float32
i32 = jnp.int32

N0 = 10000
N0P = 10240
E = 160000
EP = 163840
H = 64
NC, NS = 2, 16
NW = NC * NS
EPW = EP // NW
EHALF = EP // NC
GC = 1024
RPT = N0P // NS
BLKE = 4096

_mesh = plsc.VectorSubcoreMesh(core_axis_name="c", subcore_axis_name="s")


def _silu(x):
    return x * jax.nn.sigmoid(x)


GC2 = 512
NCH = EPW // GC2


def _make_gather2(width, dtype):
    @functools.partial(
        pl.kernel,
        out_type=jax.ShapeDtypeStruct((EP, 2 * width), dtype),
        mesh=_mesh,
        scratch_types=[
            pltpu.VMEM_SHARED((N0P, width), dtype),
            pltpu.VMEM((EPW,), i32),
            pltpu.VMEM((EPW,), i32),
            pltpu.VMEM((GC2, width), dtype),
            pltpu.VMEM((GC2, width), dtype),
            pltpu.SemaphoreType.DMA,
            pltpu.SemaphoreType.DMA,
            pltpu.SemaphoreType.DMA,
            pltpu.SemaphoreType.DMA,
        ],
        compiler_params=pltpu.CompilerParams(use_tc_tiling_on_sc=False),
    )
    def _g2(h_h, s_h, r_h, xx_h, shared, sv, rv, row0, row1,
            g0, g1, w0, w1):
        s = lax.axis_index("s")
        wid = s * NC + lax.axis_index("c")
        base = wid * EPW
        pltpu.sync_copy(h_h.at[pl.ds(s * RPT, RPT)],
                        shared.at[pl.ds(s * RPT, RPT)])
        pltpu.sync_copy(s_h.at[pl.ds(base, EPW)], sv)
        pltpu.sync_copy(r_h.at[pl.ds(base, EPW)], rv)
        plsc.subcore_barrier()

        rows = (row0, row1)
        gsems = (g0, g1)
        wsems = (w0, w1)
        nsteps = 2 * NCH
        wdesc = [None] * nsteps
        for k in range(nsteps):
            b = k % 2
            if k >= 2:
                wdesc[k - 2].wait()
            iv = sv if k < NCH else rv
            col = 0 if k < NCH else width
            j = k % NCH
            pltpu.async_copy(shared.at[iv.at[pl.ds(j * GC2, GC2)]],
                             rows[b], gsems[b]).wait()
            wdesc[k] = pltpu.async_copy(
                rows[b],
                xx_h.at[pl.ds(base + j * GC2, GC2), pl.ds(col, width)],
                wsems[b])
        wdesc[nsteps - 2].wait()
        wdesc[nsteps - 1].wait()

    return _g2


_sc_gather2 = _make_gather2(H, f32)


@functools.partial(
    pl.kernel,
    out_type=(jax.ShapeDtypeStruct((EP, 16), f32),
              jax.ShapeDtypeStruct((EP, 16), f32)),
    mesh=_mesh,
    scratch_types=[
        pltpu.VMEM_SHARED((N0P, 16), f32),
        pltpu.VMEM((EPW,), i32),
        pltpu.VMEM((EPW,), i32),
        pltpu.VMEM((GC2, 16), f32),
        pltpu.VMEM((GC2, 16), f32),
        pltpu.SemaphoreType.DMA,
        pltpu.SemaphoreType.DMA,
        pltpu.SemaphoreType.DMA,
        pltpu.SemaphoreType.DMA,
    ],
    compiler_params=pltpu.CompilerParams(use_tc_tiling_on_sc=False),
)
def _sc_gather2_pos(h_h, s_h, r_h, o1_h, o2_h, shared, sv, rv, row0, row1,
                    g0, g1, w0, w1):
    s = lax.axis_index("s")
    wid = s * NC + lax.axis_index("c")
    base = wid * EPW
    pltpu.sync_copy(h_h.at[pl.ds(s * RPT, RPT)],
                    shared.at[pl.ds(s * RPT, RPT)])
    pltpu.sync_copy(s_h.at[pl.ds(base, EPW)], sv)
    pltpu.sync_copy(r_h.at[pl.ds(base, EPW)], rv)
    plsc.subcore_barrier()

    rows = (row0, row1)
    gsems = (g0, g1)
    wsems = (w0, w1)
    nsteps = 2 * NCH
    wdesc = [None] * nsteps
    for k in range(nsteps):
        b = k % 2
        if k >= 2:
            wdesc[k - 2].wait()
        iv = sv if k < NCH else rv
        oh = o1_h if k < NCH else o2_h
        j = k % NCH
        pltpu.async_copy(shared.at[iv.at[pl.ds(j * GC2, GC2)]],
                         rows[b], gsems[b]).wait()
        wdesc[k] = pltpu.async_copy(
            rows[b], oh.at[pl.ds(base + j * GC2, GC2)], wsems[b])
    wdesc[nsteps - 2].wait()
    wdesc[nsteps - 1].wait()


@functools.partial(
    pl.kernel,
    out_type=(jax.ShapeDtypeStruct((EP, 2 * H), f32),
              jax.ShapeDtypeStruct((EP, 8), f32),
              jax.ShapeDtypeStruct((EP, 8), f32)),
    mesh=_mesh,
    scratch_types=[
        pltpu.VMEM_SHARED((N0P, H), f32),
        pltpu.VMEM_SHARED((N0P, 8), f32),
        pltpu.VMEM((EPW,), i32),
        pltpu.VMEM((EPW,), i32),
        pltpu.VMEM((GC2, H), f32),
        pltpu.VMEM((GC2, H), f32),
        pltpu.VMEM((GC2, 8), f32),
        pltpu.SemaphoreType.DMA,
        pltpu.SemaphoreType.DMA,
        pltpu.SemaphoreType.DMA,
        pltpu.SemaphoreType.DMA,
        pltpu.SemaphoreType.DMA,
    ],
    compiler_params=pltpu.CompilerParams(use_tc_tiling_on_sc=False),
)
def _sc_gather_l1(h_h, p_h, s_h, r_h, xx_h, po1_h, po2_h, sh, sp, sv, rv,
                  rh0, rh1, rowp0, gh0, gh1, wh0, wh1, gp0):
    s = lax.axis_index("s")
    wid = s * NC + lax.axis_index("c")
    base = wid * EPW
    pltpu.sync_copy(h_h.at[pl.ds(s * RPT, RPT)], sh.at[pl.ds(s * RPT, RPT)])
    pltpu.sync_copy(p_h.at[pl.ds(s * RPT, RPT)], sp.at[pl.ds(s * RPT, RPT)])
    pltpu.sync_copy(s_h.at[pl.ds(base, EPW)], sv)
    pltpu.sync_copy(r_h.at[pl.ds(base, EPW)], rv)
    plsc.subcore_barrier()

    rowh = (rh0, rh1)
    ghs = (gh0, gh1)
    whs = (wh0, wh1)
    nsteps = 2 * NCH
    wdh = [None] * nsteps
    for k in range(nsteps):
        b = k % 2
        if k >= 2:
            wdh[k - 2].wait()
        iv = sv if k < NCH else rv
        col = 0 if k < NCH else H
        po = po1_h if k < NCH else po2_h
        j = k % NCH
        idx = iv.at[pl.ds(j * GC2, GC2)]
        pltpu.async_copy(sh.at[idx], rowh[b], ghs[b]).wait()
        pltpu.async_copy(sp.at[idx], rowp0, gp0).wait()
        wdh[k] = pltpu.async_copy(
            rowh[b],
            xx_h.at[pl.ds(base + j * GC2, GC2), pl.ds(col, H)], whs[b])
        pltpu.sync_copy(rowp0, po.at[pl.ds(base + j * GC2, GC2)])
    wdh[nsteps - 2].wait()
    wdh[nsteps - 1].wait()


EP2 = EP // 2
EPW2 = EP2 // NW
GCH = 512


@functools.partial(
    pl.kernel,
    out_type=jax.ShapeDtypeStruct((NC, N0P, H), f32),
    mesh=_mesh,
    scratch_types=[
        pltpu.VMEM_SHARED((N0P, H), f32),
        pltpu.VMEM((GCH,), i32),
        pltpu.VMEM((GCH,), i32),
        pltpu.VMEM((GCH, H), f32),
        pltpu.VMEM((GCH, H), f32),
    ],
    compiler_params=pltpu.CompilerParams(use_tc_tiling_on_sc=False),
)
def _sc_scatter(m_h, re_h, ro_h, z_h, o_h, shared, ive, ivo, mva, mvb):
    c = lax.axis_index("c")
    s = lax.axis_index("s")
    pltpu.sync_copy(z_h.at[pl.ds(s * RPT, RPT)], shared.at[pl.ds(s * RPT, RPT)])
    plsc.subcore_barrier()
    ebase = c * (EP2 // NC) + s * EPW2

    def chunk(j, carry):
        off = ebase + j * GCH
        pltpu.sync_copy(re_h.at[pl.ds(off, GCH)], ive)
        pltpu.sync_copy(ro_h.at[pl.ds(off, GCH)], ivo)
        pltpu.sync_copy(m_h.at[pl.ds(off, GCH), pl.ds(0, H)], mva)
        pltpu.sync_copy(m_h.at[pl.ds(off, GCH), pl.ds(H, H)], mvb)
        pltpu.sync_copy(mva, shared.at[ive], add=True)
        pltpu.sync_copy(mvb, shared.at[ivo], add=True)
        return carry

    lax.fori_loop(0, EPW2 // GCH, chunk, 0)
    plsc.subcore_barrier()
    pltpu.sync_copy(shared.at[pl.ds(s * RPT, RPT)],
                    o_h.at[c, pl.ds(s * RPT, RPT)])


def _emb_body(x_ref, w_ref, b_ref, o_ref):
    o_ref[...] = jnp.dot(x_ref[...], w_ref[...],
                         preferred_element_type=f32) + b_ref[...]


def _tc_emb(x, w, b):
    return pl.pallas_call(
        _emb_body,
        out_shape=jax.ShapeDtypeStruct((N0P, H), f32),
    )(x, w, b[None, :])


_SEL = np.zeros((128, 16), np.float32)
for _g in range(16):
    _SEL[8 * _g:8 * _g + 3, _g] = 1.0
BLKD = 2048


def _d2_body(p1_ref, p2_ref, sel_ref, o_ref, s1_ref, s2_ref):
    i = pl.program_id(0)
    diff = p1_ref[...] - p2_ref[...]
    d2 = jnp.dot(diff * diff, sel_ref[...], preferred_element_type=f32)
    dd = jnp.sqrt(d2)
    o_ref[...] = dd

    @pl.when(i == 0)
    def _init():
        s1_ref[...] = jnp.zeros_like(s1_ref)
        s2_ref[...] = jnp.zeros_like(s2_ref)

    s1_ref[...] += jnp.sum(dd, axis=0, keepdims=True)
    s2_ref[...] += jnp.sum(d2, axis=0, keepdims=True)


def _tc_d2(p1, p2, sel):
    n = (EP // 16) // BLKD
    pblk = lambda: pl.BlockSpec((BLKD, 128), lambda i: (i, 0))
    zblk = lambda c: pl.BlockSpec((1, c), lambda i: (0, 0))
    return pl.pallas_call(
        _d2_body,
        grid=(n,),
        in_specs=[pblk(), pblk(), pl.BlockSpec((128, 16), lambda i: (0, 0))],
        out_specs=(pl.BlockSpec((BLKD, 16), lambda i: (i, 0)),
                   zblk(16), zblk(16)),
        out_shape=(jax.ShapeDtypeStruct((EP // 16, 16), f32),
                   jax.ShapeDtypeStruct((1, 16), f32),
                   jax.ShapeDtypeStruct((1, 16), f32)),
    )(p1, p2, sel)


def _msg_body(xlo_ref, xhi_ref, dlo_ref, dhi_ref, w1_ref, cp_ref, b1_ref,
              w2_ref, b2_ref, o_ref):
    def mlp(xx, dd):
        pre = (jnp.dot(xx.astype(jnp.bfloat16), w1_ref[...],
                       preferred_element_type=f32)
               + dd * cp_ref[...] + b1_ref[...])
        h1 = _silu(pre)
        return _silu(jnp.dot(h1.astype(jnp.bfloat16), w2_ref[...],
                             preferred_element_type=f32) + b2_ref[...])

    m_lo = mlp(xlo_ref[...], dlo_ref[...])
    m_hi = mlp(xhi_ref[...], dhi_ref[...])
    o_ref[...] = jnp.concatenate([m_lo, m_hi], axis=1)


def _tc_msg(xx, dcol, w1, cp, b1, w2, b2):
    n = EP2 // BLKE
    wblk = lambda r: pl.BlockSpec((r, H), lambda i: (0, 0))
    return pl.pallas_call(
        _msg_body,
        grid=(n,),
        in_specs=[pl.BlockSpec((BLKE, 2 * H), lambda i: (i, 0)),
                  pl.BlockSpec((BLKE, 2 * H), lambda i: (i + n, 0)),
                  pl.BlockSpec((BLKE, 1), lambda i: (i, 0)),
                  pl.BlockSpec((BLKE, 1), lambda i: (i + n, 0)),
                  pl.BlockSpec((2 * H, H), lambda i: (0, 0)),
                  wblk(1), wblk(1), wblk(H), wblk(1)],
        out_specs=pl.BlockSpec((BLKE, 2 * H), lambda i: (i, 0)),
        out_shape=jax.ShapeDtypeStruct((EP2, 2 * H), f32),
    )(xx, xx, dcol, dcol, w1, cp[None, :], b1[None, :], w2, b2[None, :])


def _upd_body(h_ref, a0_ref, a1_ref, u1x_ref, u1a_ref, bu1_ref,
              u2_ref, bu2_ref, o_ref):
    hh = h_ref[...]
    agg = a0_ref[...] + a1_ref[...]
    u = _silu(jnp.dot(hh, u1x_ref[...], preferred_element_type=f32)
              + jnp.dot(agg, u1a_ref[...], preferred_element_type=f32)
              + bu1_ref[...])
    o_ref[...] = hh + _silu(jnp.dot(u, u2_ref[...],
                                    preferred_element_type=f32) + bu2_ref[...])


def _tc_upd(h, a0, a1, u1x, u1a, bu1, u2, bu2):
    return pl.pallas_call(
        _upd_body,
        out_shape=jax.ShapeDtypeStruct((N0P, H), f32),
    )(h, a0, a1, u1x, u1a, bu1[None, :], u2, bu2[None, :])


def _upd_final_body(h_ref, a0_ref, a1_ref, u1x_ref, u1a_ref, bu1_ref,
                    u2_ref, bu2_ref, p_ref, bp_ref, o_ref):
    hh = h_ref[...]
    agg = a0_ref[...] + a1_ref[...]
    u = _silu(jnp.dot(hh, u1x_ref[...], preferred_element_type=f32)
              + jnp.dot(agg, u1a_ref[...], preferred_element_type=f32)
              + bu1_ref[...])
    h2 = hh + _silu(jnp.dot(u, u2_ref[...],
                            preferred_element_type=f32) + bu2_ref[...])
    o_ref[...] = jnp.dot(h2, p_ref[...],
                         preferred_element_type=f32) + bp_ref[...]


def _tc_upd_final(h, a0, a1, u1x, u1a, bu1, u2, bu2, p, bp):
    return pl.pallas_call(
        _upd_final_body,
        out_shape=jax.ShapeDtypeStruct((N0P, H), f32),
    )(h, a0, a1, u1x, u1a, bu1[None, :], u2, bu2[None, :], p, bp[None, :])


def kernel(x_0, x_1, pos, cell_ind_0, cell_ind_1, adj_0_0, adj_0_1_send,
           adj_0_1_recv, adj_1_1, params):
    send = adj_0_0[0].astype(i32)
    recv = adj_0_0[1].astype(i32)
    pad = EP - E
    zi = jnp.zeros((pad,), i32)
    send_p = jnp.concatenate([send, zi])
    recv_p = jnp.concatenate([recv, zi])
    recv_sc = jnp.concatenate([recv, N0 + (jnp.arange(pad, dtype=i32) % 8)])

    posrow = jnp.pad(pos, ((0, N0P - N0), (0, 5)))
    x0p = jnp.pad(x_0, ((0, N0P - N0), (0, 0)))

    h = _tc_emb(x0p, params["emb"]["0"]["W"], params["emb"]["0"]["b"])

    xx1, po1, po2 = _sc_gather_l1(h, posrow, send_p, recv_p)
    d16, s1, s2 = _tc_d2(po1.reshape(EP // 16, 128),
                         po2.reshape(EP // 16, 128), jnp.asarray(_SEL))
    dcol = d16.reshape(EP, 1)
    mu = jnp.sum(s1) / E
    var = jnp.sum(s2) / E - mu * mu
    sinv = lax.rsqrt(var + 1e-5)

    zeros_n = jnp.zeros((N0P, H), f32)

    nlayers = len(params["layers"])
    for li, layer in enumerate(params["layers"]):
        w1 = layer["msg"]["0_0"]["l1"]["W"]
        b1 = layer["msg"]["0_0"]["l1"]["b"]
        c_w = w1[2 * H:]
        ctil = jnp.sum(c_w, axis=0)
        cp = ctil * sinv
        b1p = b1 - mu * sinv * ctil
        w2 = layer["msg"]["0_0"]["l2"]["W"]
        b2 = layer["msg"]["0_0"]["l2"]["b"]

        xx = xx1 if li == 0 else _sc_gather2(h, send_p, recv_p)
        m2 = _tc_msg(xx, dcol, w1[:2 * H].astype(jnp.bfloat16), cp, b1p,
                     w2.astype(jnp.bfloat16), b2)
        aggp = _sc_scatter(m2, recv_sc[:EP2], recv_sc[EP2:], zeros_n)

        u1 = layer["upd"]["0"]["l1"]["W"]
        bu1 = layer["upd"]["0"]["l1"]["b"]
        u2 = layer["upd"]["0"]["l2"]["W"]
        bu2 = layer["upd"]["0"]["l2"]["b"]
        if li == nlayers - 1:
            h = _tc_upd_final(h, aggp[0], aggp[1], u1[:H], u1[H:], bu1,
                              u2, bu2, params["pre_pool"]["0"]["W"],
                              params["pre_pool"]["0"]["b"])
        else:
            h = _tc_upd(h, aggp[0], aggp[1], u1[:H], u1[H:], bu1, u2, bu2)

    out = h[:N0]
    return (out, out, jnp.zeros((1, H), f32))


# speedup vs baseline: 11.3663x; 1.1576x over previous
import functools

import numpy as np

import jax
import jax.numpy as jnp
from jax import lax
from jax.experimental import pallas as pl
from jax.experimental.pallas import tpu as pltpu
from jax.experimental.pallas import tpu_sc as plsc

f32 = jnp.float32
i32 = jnp.int32

N0 = 10000
N0P = 10240
E = 160000
EP = 163840
H = 64
NC, NS = 2, 16
NW = NC * NS
EPW = EP // NW
EHALF = EP // NC
GC = 1024
RPT = N0P // NS
BLKE = 4096

_mesh = plsc.VectorSubcoreMesh(core_axis_name="c", subcore_axis_name="s")


def _silu(x):
    return x * jax.nn.sigmoid(x)


GC2 = 512
NCH = EPW // GC2


def _make_gather2(width, dtype):
    @functools.partial(
        pl.kernel,
        out_type=jax.ShapeDtypeStruct((EP, 2 * width), dtype),
        mesh=_mesh,
        scratch_types=[
            pltpu.VMEM_SHARED((N0P, width), dtype),
            pltpu.VMEM((EPW,), i32),
            pltpu.VMEM((EPW,), i32),
            pltpu.VMEM((GC2, width), dtype),
            pltpu.VMEM((GC2, width), dtype),
            pltpu.SemaphoreType.DMA,
            pltpu.SemaphoreType.DMA,
            pltpu.SemaphoreType.DMA,
            pltpu.SemaphoreType.DMA,
        ],
        compiler_params=pltpu.CompilerParams(use_tc_tiling_on_sc=False),
    )
    def _g2(h_h, s_h, r_h, xx_h, shared, sv, rv, row0, row1,
            g0, g1, w0, w1):
        s = lax.axis_index("s")
        wid = s * NC + lax.axis_index("c")
        base = wid * EPW
        pltpu.sync_copy(h_h.at[pl.ds(s * RPT, RPT)],
                        shared.at[pl.ds(s * RPT, RPT)])
        pltpu.sync_copy(s_h.at[pl.ds(base, EPW)], sv)
        pltpu.sync_copy(r_h.at[pl.ds(base, EPW)], rv)
        plsc.subcore_barrier()

        rows = (row0, row1)
        gsems = (g0, g1)
        wsems = (w0, w1)
        nsteps = 2 * NCH
        wdesc = [None] * nsteps
        for k in range(nsteps):
            b = k % 2
            if k >= 2:
                wdesc[k - 2].wait()
            iv = sv if k < NCH else rv
            col = 0 if k < NCH else width
            j = k % NCH
            pltpu.async_copy(shared.at[iv.at[pl.ds(j * GC2, GC2)]],
                             rows[b], gsems[b]).wait()
            wdesc[k] = pltpu.async_copy(
                rows[b],
                xx_h.at[pl.ds(base + j * GC2, GC2), pl.ds(col, width)],
                wsems[b])
        wdesc[nsteps - 2].wait()
        wdesc[nsteps - 1].wait()

    return _g2


_sc_gather2 = _make_gather2(H, f32)


@functools.partial(
    pl.kernel,
    out_type=(jax.ShapeDtypeStruct((EP, 16), f32),
              jax.ShapeDtypeStruct((EP, 16), f32)),
    mesh=_mesh,
    scratch_types=[
        pltpu.VMEM_SHARED((N0P, 16), f32),
        pltpu.VMEM((EPW,), i32),
        pltpu.VMEM((EPW,), i32),
        pltpu.VMEM((GC2, 16), f32),
        pltpu.VMEM((GC2, 16), f32),
        pltpu.SemaphoreType.DMA,
        pltpu.SemaphoreType.DMA,
        pltpu.SemaphoreType.DMA,
        pltpu.SemaphoreType.DMA,
    ],
    compiler_params=pltpu.CompilerParams(use_tc_tiling_on_sc=False),
)
def _sc_gather2_pos(h_h, s_h, r_h, o1_h, o2_h, shared, sv, rv, row0, row1,
                    g0, g1, w0, w1):
    s = lax.axis_index("s")
    wid = s * NC + lax.axis_index("c")
    base = wid * EPW
    pltpu.sync_copy(h_h.at[pl.ds(s * RPT, RPT)],
                    shared.at[pl.ds(s * RPT, RPT)])
    pltpu.sync_copy(s_h.at[pl.ds(base, EPW)], sv)
    pltpu.sync_copy(r_h.at[pl.ds(base, EPW)], rv)
    plsc.subcore_barrier()

    rows = (row0, row1)
    gsems = (g0, g1)
    wsems = (w0, w1)
    nsteps = 2 * NCH
    wdesc = [None] * nsteps
    for k in range(nsteps):
        b = k % 2
        if k >= 2:
            wdesc[k - 2].wait()
        iv = sv if k < NCH else rv
        oh = o1_h if k < NCH else o2_h
        j = k % NCH
        pltpu.async_copy(shared.at[iv.at[pl.ds(j * GC2, GC2)]],
                         rows[b], gsems[b]).wait()
        wdesc[k] = pltpu.async_copy(
            rows[b], oh.at[pl.ds(base + j * GC2, GC2)], wsems[b])
    wdesc[nsteps - 2].wait()
    wdesc[nsteps - 1].wait()


EP2 = EP // 2
EPW2 = EP2 // NW
GCH = 512


@functools.partial(
    pl.kernel,
    out_type=jax.ShapeDtypeStruct((N0P, 2 * H), f32),
    mesh=_mesh,
    scratch_types=[
        pltpu.VMEM_SHARED((N0P, H), f32),
        pltpu.VMEM((GCH,), i32),
        pltpu.VMEM((GCH,), i32),
        pltpu.VMEM((GCH, H), f32),
        pltpu.VMEM((GCH, H), f32),
    ],
    compiler_params=pltpu.CompilerParams(use_tc_tiling_on_sc=False),
)
def _sc_scatter(m_h, re_h, ro_h, z_h, o_h, shared, ive, ivo, mva, mvb):
    c = lax.axis_index("c")
    s = lax.axis_index("s")
    pltpu.sync_copy(z_h.at[pl.ds(s * RPT, RPT)], shared.at[pl.ds(s * RPT, RPT)])
    plsc.subcore_barrier()
    ebase = c * (EP2 // NC) + s * EPW2

    def chunk(j, carry):
        off = ebase + j * GCH
        pltpu.sync_copy(re_h.at[pl.ds(off, GCH)], ive)
        pltpu.sync_copy(ro_h.at[pl.ds(off, GCH)], ivo)
        pltpu.sync_copy(m_h.at[pl.ds(off, GCH), pl.ds(0, H)], mva)
        pltpu.sync_copy(m_h.at[pl.ds(off, GCH), pl.ds(H, H)], mvb)
        pltpu.sync_copy(mva, shared.at[ive], add=True)
        pltpu.sync_copy(mvb, shared.at[ivo], add=True)
        return carry

    lax.fori_loop(0, EPW2 // GCH, chunk, 0)
    plsc.subcore_barrier()
    pltpu.sync_copy(shared.at[pl.ds(s * RPT, RPT)],
                    o_h.at[pl.ds(s * RPT, RPT), pl.ds(c * H, H)])


def _emb_body(x_ref, w_ref, b_ref, o_ref):
    o_ref[...] = jnp.dot(x_ref[...], w_ref[...],
                         preferred_element_type=f32) + b_ref[...]


def _tc_emb(x, w, b):
    return pl.pallas_call(
        _emb_body,
        out_shape=jax.ShapeDtypeStruct((N0P, H), f32),
    )(x, w, b[None, :])


_SEL = np.zeros((128, 8), np.float32)
for _g in range(8):
    _SEL[16 * _g:16 * _g + 3, _g] = 1.0


def _d2_body(p1_ref, p2_ref, sel_ref, o_ref, s1_ref, s2_ref):
    i = pl.program_id(0)
    diff = p1_ref[...] - p2_ref[...]
    d2 = jnp.dot(diff * diff, sel_ref[...], preferred_element_type=f32)
    dd = jnp.sqrt(d2)
    o_ref[...] = dd

    @pl.when(i == 0)
    def _init():
        s1_ref[...] = jnp.zeros_like(s1_ref)
        s2_ref[...] = jnp.zeros_like(s2_ref)

    s1_ref[...] += jnp.sum(dd, axis=0, keepdims=True)
    s2_ref[...] += jnp.sum(d2, axis=0, keepdims=True)


def _tc_d2(p1, p2, sel):
    n = (EP // 8) // BLKE
    pblk = lambda: pl.BlockSpec((BLKE, 128), lambda i: (i, 0))
    zblk = lambda c: pl.BlockSpec((1, c), lambda i: (0, 0))
    return pl.pallas_call(
        _d2_body,
        grid=(n,),
        in_specs=[pblk(), pblk(), pl.BlockSpec((128, 8), lambda i: (0, 0))],
        out_specs=(pl.BlockSpec((BLKE, 8), lambda i: (i, 0)),
                   zblk(8), zblk(8)),
        out_shape=(jax.ShapeDtypeStruct((EP // 8, 8), f32),
                   jax.ShapeDtypeStruct((1, 8), f32),
                   jax.ShapeDtypeStruct((1, 8), f32)),
    )(p1, p2, sel)


def _msg_body(xlo_ref, xhi_ref, dlo_ref, dhi_ref, w1_ref, cp_ref, b1_ref,
              w2_ref, b2_ref, o_ref):
    def mlp(xx, dd):
        pre = (jnp.dot(xx.astype(jnp.bfloat16), w1_ref[...],
                       preferred_element_type=f32)
               + dd * cp_ref[...] + b1_ref[...])
        h1 = _silu(pre)
        return _silu(jnp.dot(h1.astype(jnp.bfloat16), w2_ref[...],
                             preferred_element_type=f32) + b2_ref[...])

    m_lo = mlp(xlo_ref[...], dlo_ref[...])
    m_hi = mlp(xhi_ref[...], dhi_ref[...])
    o_ref[...] = jnp.concatenate([m_lo, m_hi], axis=1)


def _tc_msg(xx, dcol, w1, cp, b1, w2, b2):
    n = EP2 // BLKE
    wblk = lambda r: pl.BlockSpec((r, H), lambda i: (0, 0))
    return pl.pallas_call(
        _msg_body,
        grid=(n,),
        in_specs=[pl.BlockSpec((BLKE, 2 * H), lambda i: (i, 0)),
                  pl.BlockSpec((BLKE, 2 * H), lambda i: (i + n, 0)),
                  pl.BlockSpec((BLKE, 1), lambda i: (i, 0)),
                  pl.BlockSpec((BLKE, 1), lambda i: (i + n, 0)),
                  pl.BlockSpec((2 * H, H), lambda i: (0, 0)),
                  wblk(1), wblk(1), wblk(H), wblk(1)],
        out_specs=pl.BlockSpec((BLKE, 2 * H), lambda i: (i, 0)),
        out_shape=jax.ShapeDtypeStruct((EP2, 2 * H), f32),
    )(xx, xx, dcol, dcol, w1, cp[None, :], b1[None, :], w2, b2[None, :])


def _upd_body(h_ref, ap_ref, u1x_ref, u1a_ref, bu1_ref,
              u2_ref, bu2_ref, o_ref):
    hh = h_ref[...]
    ap = ap_ref[...]
    agg = ap[:, 0:H] + ap[:, H:2 * H]
    u = _silu(jnp.dot(hh, u1x_ref[...], preferred_element_type=f32)
              + jnp.dot(agg, u1a_ref[...], preferred_element_type=f32)
              + bu1_ref[...])
    o_ref[...] = hh + _silu(jnp.dot(u, u2_ref[...],
                                    preferred_element_type=f32) + bu2_ref[...])


def _tc_upd(h, ap, u1x, u1a, bu1, u2, bu2):
    return pl.pallas_call(
        _upd_body,
        out_shape=jax.ShapeDtypeStruct((N0P, H), f32),
    )(h, ap, u1x, u1a, bu1[None, :], u2, bu2[None, :])


def _upd_final_body(h_ref, ap_ref, u1x_ref, u1a_ref, bu1_ref,
                    u2_ref, bu2_ref, p_ref, bp_ref, o_ref):
    hh = h_ref[...]
    ap = ap_ref[...]
    agg = ap[:, 0:H] + ap[:, H:2 * H]
    u = _silu(jnp.dot(hh, u1x_ref[...], preferred_element_type=f32)
              + jnp.dot(agg, u1a_ref[...], preferred_element_type=f32)
              + bu1_ref[...])
    h2 = hh + _silu(jnp.dot(u, u2_ref[...],
                            preferred_element_type=f32) + bu2_ref[...])
    o_ref[...] = jnp.dot(h2, p_ref[...],
                         preferred_element_type=f32) + bp_ref[...]


def _tc_upd_final(h, ap, u1x, u1a, bu1, u2, bu2, p, bp):
    return pl.pallas_call(
        _upd_final_body,
        out_shape=jax.ShapeDtypeStruct((N0P, H), f32),
    )(h, ap, u1x, u1a, bu1[None, :], u2, bu2[None, :], p, bp[None, :])


def kernel(x_0, x_1, pos, cell_ind_0, cell_ind_1, adj_0_0, adj_0_1_send,
           adj_0_1_recv, adj_1_1, params):
    send = adj_0_0[0].astype(i32)
    recv = adj_0_0[1].astype(i32)
    pad = EP - E
    zi = jnp.zeros((pad,), i32)
    send_p = jnp.concatenate([send, zi])
    recv_p = jnp.concatenate([recv, zi])
    recv_sc = jnp.concatenate([recv, N0 + (jnp.arange(pad, dtype=i32) % 8)])

    posrow = jnp.pad(pos, ((0, N0P - N0), (0, 13)))
    x0p = jnp.pad(x_0, ((0, N0P - N0), (0, 0)))

    h = _tc_emb(x0p, params["emb"]["0"]["W"], params["emb"]["0"]["b"])

    po1, po2 = _sc_gather2_pos(posrow, send_p, recv_p)
    d8, s1, s2 = _tc_d2(po1.reshape(EP // 8, 128), po2.reshape(EP // 8, 128),
                        jnp.asarray(_SEL))
    dcol = d8.reshape(EP, 1)
    mu = jnp.sum(s1) / E
    var = jnp.sum(s2) / E - mu * mu
    sinv = lax.rsqrt(var + 1e-5)

    zeros_n = jnp.zeros((N0P, H), f32)

    nlayers = len(params["layers"])
    for li, layer in enumerate(params["layers"]):
        w1 = layer["msg"]["0_0"]["l1"]["W"]
        b1 = layer["msg"]["0_0"]["l1"]["b"]
        c_w = w1[2 * H:]
        ctil = jnp.sum(c_w, axis=0)
        cp = ctil * sinv
        b1p = b1 - mu * sinv * ctil
        w2 = layer["msg"]["0_0"]["l2"]["W"]
        b2 = layer["msg"]["0_0"]["l2"]["b"]

        xx = _sc_gather2(h, send_p, recv_p)
        m2 = _tc_msg(xx, dcol, w1[:2 * H].astype(jnp.bfloat16), cp, b1p,
                     w2.astype(jnp.bfloat16), b2)
        aggp = _sc_scatter(m2, recv_sc[:EP2], recv_sc[EP2:], zeros_n)

        u1 = layer["upd"]["0"]["l1"]["W"]
        bu1 = layer["upd"]["0"]["l1"]["b"]
        u2 = layer["upd"]["0"]["l2"]["W"]
        bu2 = layer["upd"]["0"]["l2"]["b"]
        if li == nlayers - 1:
            h = _tc_upd_final(h, aggp, u1[:H], u1[H:], bu1,
                              u2, bu2, params["pre_pool"]["0"]["W"],
                              params["pre_pool"]["0"]["b"])
        else:
            h = _tc_upd(h, aggp, u1[:H], u1[H:], bu1, u2, bu2)

    out = h[:N0]
    return (out, out, jnp.zeros((1, H), f32))


# speedup vs baseline: 11.6400x; 1.0241x over previous
import functools

import numpy as np

import jax
import jax.numpy as jnp
from jax import lax
from jax.experimental import pallas as pl
from jax.experimental.pallas import tpu as pltpu
from jax.experimental.pallas import tpu_sc as plsc

f32 = jnp.float32
i32 = jnp.int32

N0 = 10000
N0P = 10240
E = 160000
EP = 163840
H = 64
NC, NS = 2, 16
NW = NC * NS
EPW = EP // NW
EHALF = EP // NC
GC = 1024
RPT = N0P // NS
BLKE = 4096

_mesh = plsc.VectorSubcoreMesh(core_axis_name="c", subcore_axis_name="s")


def _silu(x):
    return x * jax.nn.sigmoid(x)


GC2 = 512
NCH = EPW // GC2


def _make_gather2(width, dtype):
    @functools.partial(
        pl.kernel,
        out_type=jax.ShapeDtypeStruct((EP, 2 * width), dtype),
        mesh=_mesh,
        scratch_types=[
            pltpu.VMEM_SHARED((N0P, width), dtype),
            pltpu.VMEM((EPW,), i32),
            pltpu.VMEM((EPW,), i32),
            pltpu.VMEM((GC2, width), dtype),
            pltpu.VMEM((GC2, width), dtype),
            pltpu.SemaphoreType.DMA,
            pltpu.SemaphoreType.DMA,
            pltpu.SemaphoreType.DMA,
            pltpu.SemaphoreType.DMA,
        ],
        compiler_params=pltpu.CompilerParams(use_tc_tiling_on_sc=False),
    )
    def _g2(h_h, s_h, r_h, xx_h, shared, sv, rv, row0, row1,
            g0, g1, w0, w1):
        s = lax.axis_index("s")
        wid = s * NC + lax.axis_index("c")
        base = wid * EPW
        pltpu.sync_copy(h_h.at[pl.ds(s * RPT, RPT)],
                        shared.at[pl.ds(s * RPT, RPT)])
        pltpu.sync_copy(s_h.at[pl.ds(base, EPW)], sv)
        pltpu.sync_copy(r_h.at[pl.ds(base, EPW)], rv)
        plsc.subcore_barrier()

        rows = (row0, row1)
        gsems = (g0, g1)
        wsems = (w0, w1)
        nsteps = 2 * NCH

        def idx_at(k):
            iv = sv if k < NCH else rv
            return iv.at[pl.ds((k % NCH) * GC2, GC2)]

        def out_at(k):
            col = 0 if k < NCH else width
            return xx_h.at[pl.ds(base + (k % NCH) * GC2, GC2),
                           pl.ds(col, width)]

        gd = [None] * nsteps
        wdesc = [None] * nsteps
        gd[0] = pltpu.async_copy(shared.at[idx_at(0)], rows[0], gsems[0])
        for k in range(nsteps):
            b = k % 2
            if k + 1 < nsteps:
                if k >= 1:
                    wdesc[k - 1].wait()
                gd[k + 1] = pltpu.async_copy(shared.at[idx_at(k + 1)],
                                             rows[(k + 1) % 2],
                                             gsems[(k + 1) % 2])
            gd[k].wait()
            wdesc[k] = pltpu.async_copy(rows[b], out_at(k), wsems[b])
        wdesc[nsteps - 2].wait()
        wdesc[nsteps - 1].wait()

    return _g2


_sc_gather2 = _make_gather2(H, f32)


@functools.partial(
    pl.kernel,
    out_type=(jax.ShapeDtypeStruct((EP, 16), f32),
              jax.ShapeDtypeStruct((EP, 16), f32)),
    mesh=_mesh,
    scratch_types=[
        pltpu.VMEM_SHARED((N0P, 16), f32),
        pltpu.VMEM((EPW,), i32),
        pltpu.VMEM((EPW,), i32),
        pltpu.VMEM((GC2, 16), f32),
        pltpu.VMEM((GC2, 16), f32),
        pltpu.SemaphoreType.DMA,
        pltpu.SemaphoreType.DMA,
        pltpu.SemaphoreType.DMA,
        pltpu.SemaphoreType.DMA,
    ],
    compiler_params=pltpu.CompilerParams(use_tc_tiling_on_sc=False),
)
def _sc_gather2_pos(h_h, s_h, r_h, o1_h, o2_h, shared, sv, rv, row0, row1,
                    g0, g1, w0, w1):
    s = lax.axis_index("s")
    wid = s * NC + lax.axis_index("c")
    base = wid * EPW
    pltpu.sync_copy(h_h.at[pl.ds(s * RPT, RPT)],
                    shared.at[pl.ds(s * RPT, RPT)])
    pltpu.sync_copy(s_h.at[pl.ds(base, EPW)], sv)
    pltpu.sync_copy(r_h.at[pl.ds(base, EPW)], rv)
    plsc.subcore_barrier()

    rows = (row0, row1)
    gsems = (g0, g1)
    wsems = (w0, w1)
    nsteps = 2 * NCH
    wdesc = [None] * nsteps
    for k in range(nsteps):
        b = k % 2
        if k >= 2:
            wdesc[k - 2].wait()
        iv = sv if k < NCH else rv
        oh = o1_h if k < NCH else o2_h
        j = k % NCH
        pltpu.async_copy(shared.at[iv.at[pl.ds(j * GC2, GC2)]],
                         rows[b], gsems[b]).wait()
        wdesc[k] = pltpu.async_copy(
            rows[b], oh.at[pl.ds(base + j * GC2, GC2)], wsems[b])
    wdesc[nsteps - 2].wait()
    wdesc[nsteps - 1].wait()


EP2 = EP // 2
EPW2 = EP2 // NW
GCH = 512


@functools.partial(
    pl.kernel,
    out_type=jax.ShapeDtypeStruct((N0P, 2 * H), f32),
    mesh=_mesh,
    scratch_types=[
        pltpu.VMEM_SHARED((N0P, H), f32),
        pltpu.VMEM((GCH,), i32),
        pltpu.VMEM((GCH,), i32),
        pltpu.VMEM((GCH, H), f32),
        pltpu.VMEM((GCH, H), f32),
        pltpu.SemaphoreType.DMA,
        pltpu.SemaphoreType.DMA,
        pltpu.SemaphoreType.DMA,
        pltpu.SemaphoreType.DMA,
    ],
    compiler_params=pltpu.CompilerParams(use_tc_tiling_on_sc=False),
)
def _sc_scatter(m_h, re_h, ro_h, z_h, o_h, shared, ive, ivo, mva, mvb,
                se, so, sa, sb):
    c = lax.axis_index("c")
    s = lax.axis_index("s")
    pltpu.sync_copy(z_h.at[pl.ds(s * RPT, RPT)], shared.at[pl.ds(s * RPT, RPT)])
    plsc.subcore_barrier()
    ebase = c * (EP2 // NC) + s * EPW2

    def chunk(j, carry):
        off = ebase + j * GCH
        de = pltpu.async_copy(re_h.at[pl.ds(off, GCH)], ive, se)
        do = pltpu.async_copy(ro_h.at[pl.ds(off, GCH)], ivo, so)
        da = pltpu.async_copy(m_h.at[pl.ds(off, GCH), pl.ds(0, H)], mva, sa)
        db = pltpu.async_copy(m_h.at[pl.ds(off, GCH), pl.ds(H, H)], mvb, sb)
        de.wait()
        da.wait()
        pltpu.sync_copy(mva, shared.at[ive], add=True)
        do.wait()
        db.wait()
        pltpu.sync_copy(mvb, shared.at[ivo], add=True)
        return carry

    lax.fori_loop(0, EPW2 // GCH, chunk, 0)
    plsc.subcore_barrier()
    pltpu.sync_copy(shared.at[pl.ds(s * RPT, RPT)],
                    o_h.at[pl.ds(s * RPT, RPT), pl.ds(c * H, H)])


def _emb_body(x_ref, w_ref, b_ref, o_ref):
    o_ref[...] = jnp.dot(x_ref[...], w_ref[...],
                         preferred_element_type=f32) + b_ref[...]


def _tc_emb(x, w, b):
    return pl.pallas_call(
        _emb_body,
        out_shape=jax.ShapeDtypeStruct((N0P, H), f32),
    )(x, w, b[None, :])


_SEL = np.zeros((128, 8), np.float32)
for _g in range(8):
    _SEL[16 * _g:16 * _g + 3, _g] = 1.0


def _d2_body(p1_ref, p2_ref, sel_ref, o_ref, s1_ref, s2_ref):
    i = pl.program_id(0)
    diff = p1_ref[...] - p2_ref[...]
    d2 = jnp.dot(diff * diff, sel_ref[...], preferred_element_type=f32)
    dd = jnp.sqrt(d2)
    o_ref[...] = dd

    @pl.when(i == 0)
    def _init():
        s1_ref[...] = jnp.zeros_like(s1_ref)
        s2_ref[...] = jnp.zeros_like(s2_ref)

    s1_ref[...] += jnp.sum(dd, axis=0, keepdims=True)
    s2_ref[...] += jnp.sum(d2, axis=0, keepdims=True)


def _tc_d2(p1, p2, sel):
    n = (EP // 8) // BLKE
    pblk = lambda: pl.BlockSpec((BLKE, 128), lambda i: (i, 0))
    zblk = lambda c: pl.BlockSpec((1, c), lambda i: (0, 0))
    return pl.pallas_call(
        _d2_body,
        grid=(n,),
        in_specs=[pblk(), pblk(), pl.BlockSpec((128, 8), lambda i: (0, 0))],
        out_specs=(pl.BlockSpec((BLKE, 8), lambda i: (i, 0)),
                   zblk(8), zblk(8)),
        out_shape=(jax.ShapeDtypeStruct((EP // 8, 8), f32),
                   jax.ShapeDtypeStruct((1, 8), f32),
                   jax.ShapeDtypeStruct((1, 8), f32)),
    )(p1, p2, sel)


def _msg_body(xlo_ref, xhi_ref, dlo_ref, dhi_ref, w1_ref, cp_ref, b1_ref,
              w2_ref, b2_ref, o_ref):
    def mlp(xx, dd):
        pre = (jnp.dot(xx.astype(jnp.bfloat16), w1_ref[...],
                       preferred_element_type=f32)
               + dd * cp_ref[...] + b1_ref[...])
        h1 = _silu(pre)
        return _silu(jnp.dot(h1.astype(jnp.bfloat16), w2_ref[...],
                             preferred_element_type=f32) + b2_ref[...])

    m_lo = mlp(xlo_ref[...], dlo_ref[...])
    m_hi = mlp(xhi_ref[...], dhi_ref[...])
    o_ref[...] = jnp.concatenate([m_lo, m_hi], axis=1)


def _tc_msg(xx, dcol, w1, cp, b1, w2, b2):
    n = EP2 // BLKE
    wblk = lambda r: pl.BlockSpec((r, H), lambda i: (0, 0))
    return pl.pallas_call(
        _msg_body,
        grid=(n,),
        in_specs=[pl.BlockSpec((BLKE, 2 * H), lambda i: (i, 0)),
                  pl.BlockSpec((BLKE, 2 * H), lambda i: (i + n, 0)),
                  pl.BlockSpec((BLKE, 1), lambda i: (i, 0)),
                  pl.BlockSpec((BLKE, 1), lambda i: (i + n, 0)),
                  pl.BlockSpec((2 * H, H), lambda i: (0, 0)),
                  wblk(1), wblk(1), wblk(H), wblk(1)],
        out_specs=pl.BlockSpec((BLKE, 2 * H), lambda i: (i, 0)),
        out_shape=jax.ShapeDtypeStruct((EP2, 2 * H), f32),
    )(xx, xx, dcol, dcol, w1, cp[None, :], b1[None, :], w2, b2[None, :])


def _upd_body(h_ref, ap_ref, u1x_ref, u1a_ref, bu1_ref,
              u2_ref, bu2_ref, o_ref):
    hh = h_ref[...]
    ap = ap_ref[...]
    agg = ap[:, 0:H] + ap[:, H:2 * H]
    u = _silu(jnp.dot(hh, u1x_ref[...], preferred_element_type=f32)
              + jnp.dot(agg, u1a_ref[...], preferred_element_type=f32)
              + bu1_ref[...])
    o_ref[...] = hh + _silu(jnp.dot(u, u2_ref[...],
                                    preferred_element_type=f32) + bu2_ref[...])


def _tc_upd(h, ap, u1x, u1a, bu1, u2, bu2):
    return pl.pallas_call(
        _upd_body,
        out_shape=jax.ShapeDtypeStruct((N0P, H), f32),
    )(h, ap, u1x, u1a, bu1[None, :], u2, bu2[None, :])


def _upd_final_body(h_ref, ap_ref, u1x_ref, u1a_ref, bu1_ref,
                    u2_ref, bu2_ref, p_ref, bp_ref, o_ref):
    hh = h_ref[...]
    ap = ap_ref[...]
    agg = ap[:, 0:H] + ap[:, H:2 * H]
    u = _silu(jnp.dot(hh, u1x_ref[...], preferred_element_type=f32)
              + jnp.dot(agg, u1a_ref[...], preferred_element_type=f32)
              + bu1_ref[...])
    h2 = hh + _silu(jnp.dot(u, u2_ref[...],
                            preferred_element_type=f32) + bu2_ref[...])
    o_ref[...] = jnp.dot(h2, p_ref[...],
                         preferred_element_type=f32) + bp_ref[...]


def _tc_upd_final(h, ap, u1x, u1a, bu1, u2, bu2, p, bp):
    return pl.pallas_call(
        _upd_final_body,
        out_shape=jax.ShapeDtypeStruct((N0P, H), f32),
    )(h, ap, u1x, u1a, bu1[None, :], u2, bu2[None, :], p, bp[None, :])


def kernel(x_0, x_1, pos, cell_ind_0, cell_ind_1, adj_0_0, adj_0_1_send,
           adj_0_1_recv, adj_1_1, params):
    send = adj_0_0[0].astype(i32)
    recv = adj_0_0[1].astype(i32)
    pad = EP - E
    zi = jnp.zeros((pad,), i32)
    send_p = jnp.concatenate([send, zi])
    recv_p = jnp.concatenate([recv, zi])
    recv_sc = jnp.concatenate([recv, N0 + (jnp.arange(pad, dtype=i32) % 8)])

    posrow = jnp.pad(pos, ((0, N0P - N0), (0, 13)))
    x0p = jnp.pad(x_0, ((0, N0P - N0), (0, 0)))

    h = _tc_emb(x0p, params["emb"]["0"]["W"], params["emb"]["0"]["b"])

    po1, po2 = _sc_gather2_pos(posrow, send_p, recv_p)
    d8, s1, s2 = _tc_d2(po1.reshape(EP // 8, 128), po2.reshape(EP // 8, 128),
                        jnp.asarray(_SEL))
    dcol = d8.reshape(EP, 1)
    mu = jnp.sum(s1) / E
    var = jnp.sum(s2) / E - mu * mu
    sinv = lax.rsqrt(var + 1e-5)

    zeros_n = jnp.zeros((N0P, H), f32)

    nlayers = len(params["layers"])
    for li, layer in enumerate(params["layers"]):
        w1 = layer["msg"]["0_0"]["l1"]["W"]
        b1 = layer["msg"]["0_0"]["l1"]["b"]
        c_w = w1[2 * H:]
        ctil = jnp.sum(c_w, axis=0)
        cp = ctil * sinv
        b1p = b1 - mu * sinv * ctil
        w2 = layer["msg"]["0_0"]["l2"]["W"]
        b2 = layer["msg"]["0_0"]["l2"]["b"]

        xx = _sc_gather2(h, send_p, recv_p)
        m2 = _tc_msg(xx, dcol, w1[:2 * H].astype(jnp.bfloat16), cp, b1p,
                     w2.astype(jnp.bfloat16), b2)
        aggp = _sc_scatter(m2, recv_sc[:EP2], recv_sc[EP2:], zeros_n)

        u1 = layer["upd"]["0"]["l1"]["W"]
        bu1 = layer["upd"]["0"]["l1"]["b"]
        u2 = layer["upd"]["0"]["l2"]["W"]
        bu2 = layer["upd"]["0"]["l2"]["b"]
        if li == nlayers - 1:
            h = _tc_upd_final(h, aggp, u1[:H], u1[H:], bu1,
                              u2, bu2, params["pre_pool"]["0"]["W"],
                              params["pre_pool"]["0"]["b"])
        else:
            h = _tc_upd(h, aggp, u1[:H], u1[H:], bu1, u2, bu2)

    out = h[:N0]
    return (out, out, jnp.zeros((1, H), f32))
